# baseline (speedup 1.0000x reference)
import functools

import jax
import jax.numpy as jnp
from jax import lax
from jax.experimental import pallas as pl
from jax.experimental.pallas import tpu as pltpu
from jax.experimental.pallas import tpu_sc as plsc

N = 10000
E = 320000
DN = 128
DE = 16
G = 256

NC, NS = 2, 16
TILES = NC * NS
CHUNK = 128
CPT = 80
EPT = CHUNK * CPT
E_PAD = TILES * EPT
PAD = E_PAD - E
ROWS_PT = 632
N_ACC = ROWS_PT * NS
BE = 2048
BNODE = 1000
NBLK = N // BNODE

@functools.lru_cache(maxsize=None)
def _mesh():
    return plsc.VectorSubcoreMesh(core_axis_name="c", subcore_axis_name="s",
                                  num_cores=NC, num_subcores=NS)


def _elu(v):
    return jnp.where(v > 0, v, jnp.exp(jnp.minimum(v, 0.0)) - 1.0)


def _dot(a, b):
    return jnp.dot(a, b, preferred_element_type=jnp.float32)


def _full_spec(shape):
    n = len(shape)
    return pl.BlockSpec(shape, lambda i, _n=n: (0,) * _n)


@functools.lru_cache(maxsize=None)
def _sc_gather(nf):
    nb = 2 if nf > 64 else 4
    ng = CPT // nb
    @functools.partial(
        pl.kernel,
        out_type=(jax.ShapeDtypeStruct((E_PAD, nf), jnp.float32),
                  jax.ShapeDtypeStruct((E_PAD, nf), jnp.float32)),
        mesh=_mesh(),
        scratch_types=[pltpu.VMEM((CPT, CHUNK), jnp.int32),
                       pltpu.VMEM((CPT, CHUNK), jnp.int32),
                       pltpu.VMEM((nb, CHUNK, nf), jnp.float32),
                       pltpu.VMEM((nb, CHUNK, nf), jnp.float32),
                       pltpu.SemaphoreType.DMA,
                       pltpu.SemaphoreType.DMA],
        compiler_params=pltpu.CompilerParams(use_tc_tiling_on_sc=False),
    )
    def gath(x_hbm, ridx, cidx, xr_out, xc_out, idxr, idxc, bufr, bufc,
             gsem, ssem):
        cid = lax.axis_index("c")
        sid = lax.axis_index("s")
        wid = sid * NC + cid
        base = wid * EPT
        pltpu.sync_copy(ridx.at[wid], idxr)
        pltpu.sync_copy(cidx.at[wid], idxc)

        def group(k, c):
            descs = []
            for b in range(nb):
                j = k * nb + b
                descs.append(
                    pltpu.async_copy(x_hbm.at[idxr.at[j]], bufr.at[b], gsem))
                descs.append(
                    pltpu.async_copy(x_hbm.at[idxc.at[j]], bufc.at[b], gsem))
            for d in descs:
                d.wait()
            descs = []
            for b in range(nb):
                j = k * nb + b
                dst = pl.ds(base + j * CHUNK, CHUNK)
                descs.append(
                    pltpu.async_copy(bufr.at[b], xr_out.at[dst], ssem))
                descs.append(
                    pltpu.async_copy(bufc.at[b], xc_out.at[dst], ssem))
            for d in descs:
                d.wait()
            return c

        lax.fori_loop(0, ng, group, 0)
    return gath


@functools.lru_cache(maxsize=None)
def _sc_scatter(w):
    @functools.partial(
        pl.kernel,
        out_type=(jax.ShapeDtypeStruct((N_ACC, w), jnp.float32),
                  jax.ShapeDtypeStruct((N_ACC, w), jnp.float32)),
        mesh=_mesh(),
        scratch_types=[pltpu.VMEM((CPT, CHUNK), jnp.int32),
                       pltpu.VMEM((8, CHUNK, w), jnp.float32),
                       pltpu.VMEM_SHARED((N_ACC, w), jnp.float32),
                       pltpu.SemaphoreType.DMA,
                       pltpu.SemaphoreType.DMA],
        compiler_params=pltpu.CompilerParams(use_tc_tiling_on_sc=False),
    )
    def scat(h_hbm, idx3, z_hbm, out0, out1, idxv, hbuf, acc, lsem, asem):
        cid = lax.axis_index("c")
        sid = lax.axis_index("s")
        wid = sid * NC + cid
        base = wid * EPT
        sl = pl.ds(sid * ROWS_PT, ROWS_PT)
        pltpu.sync_copy(z_hbm, acc.at[sl])
        plsc.subcore_barrier()
        pltpu.sync_copy(idx3.at[wid], idxv)
        nb = 8

        def group(k, c):
            descs = []
            for b in range(nb):
                j = k * nb + b
                descs.append(pltpu.async_copy(
                    h_hbm.at[pl.ds(base + j * CHUNK, CHUNK)], hbuf.at[b],
                    lsem))
            for d in descs:
                d.wait()
            descs = []
            for b in range(nb):
                j = k * nb + b
                descs.append(pltpu.async_copy(
                    hbuf.at[b], acc.at[idxv.at[j]], asem, add=True))
            for d in descs:
                d.wait()
            return c

        lax.fori_loop(0, CPT // nb, group, 0)
        plsc.subcore_barrier()

        @pl.when(cid == 0)
        def _():
            pltpu.sync_copy(acc.at[sl], out0.at[sl])

        @pl.when(cid == 1)
        def _():
            pltpu.sync_copy(acc.at[sl], out1.at[sl])
    return scat


@functools.lru_cache(maxsize=None)
def _sc_degree():
    @functools.partial(
        pl.kernel,
        out_type=(jax.ShapeDtypeStruct((N_ACC, 16), jnp.float32),
                  jax.ShapeDtypeStruct((N_ACC, 16), jnp.float32)),
        mesh=_mesh(),
        scratch_types=[pltpu.VMEM((CPT, CHUNK), jnp.int32),
                       pltpu.VMEM((CHUNK, 16), jnp.float32),
                       pltpu.VMEM_SHARED((N_ACC, 16), jnp.float32),
                       pltpu.SemaphoreType.DMA],
        compiler_params=pltpu.CompilerParams(use_tc_tiling_on_sc=False),
    )
    def degk(idx3, z_hbm, ones_hbm, out0, out1, idxv, obuf, acc, asem):
        cid = lax.axis_index("c")
        sid = lax.axis_index("s")
        wid = sid * NC + cid
        sl = pl.ds(sid * ROWS_PT, ROWS_PT)
        pltpu.sync_copy(z_hbm, acc.at[sl])
        pltpu.sync_copy(ones_hbm, obuf)
        plsc.subcore_barrier()
        pltpu.sync_copy(idx3.at[wid], idxv)
        nb = 8

        def group(k, c):
            descs = [pltpu.async_copy(obuf, acc.at[idxv.at[k * nb + b]],
                                      asem, add=True) for b in range(nb)]
            for d in descs:
                d.wait()
            return c

        lax.fori_loop(0, CPT // nb, group, 0)
        plsc.subcore_barrier()

        @pl.when(cid == 0)
        def _():
            pltpu.sync_copy(acc.at[sl], out0.at[sl])

        @pl.when(cid == 1)
        def _():
            pltpu.sync_copy(acc.at[sl], out1.at[sl])
    return degk


def _stats(x, bs):
    m, c = x.shape
    nb = m // bs

    def body(x_ref, o_ref):
        i = pl.program_id(0)

        @pl.when(i == 0)
        def _():
            o_ref[...] = jnp.zeros_like(o_ref)

        xb = x_ref[...]
        o_ref[0:1, :] += jnp.sum(xb, axis=0, keepdims=True)
        o_ref[1:2, :] += jnp.sum(xb * xb, axis=0, keepdims=True)

    out = pl.pallas_call(
        body,
        grid=(nb,),
        in_specs=[pl.BlockSpec((bs, c), lambda i: (i, 0))],
        out_specs=pl.BlockSpec((8, c), lambda i: (0, 0)),
        out_shape=jax.ShapeDtypeStruct((8, c), jnp.float32),
    )(x)
    return out[0], out[1]


def _edge_call(nf, ef, eo, xr, xc, e, w1a, w1b, w1c, b1, w2, b2,
               wna, wnb, bn1, wn2, bn2):
    def body(xr_ref, xc_ref, e_ref, w1a_r, w1b_r, w1c_r, b1_r, w2_r, b2_r,
             wna_r, wnb_r, bn1_r, wn2_r, bn2_r, oe_ref, oh_ref):
        xrv = xr_ref[...]
        xcv = xc_ref[...]
        ev = e_ref[...]
        t1 = _elu(_dot(xrv, w1a_r[...]) + _dot(xcv, w1b_r[...])
                  + _dot(ev, w1c_r[...]) + b1_r[...])
        en = _dot(t1, w2_r[...]) + b2_r[...]
        oe_ref[...] = en
        t2 = _elu(_dot(xrv, wna_r[...]) + _dot(en, wnb_r[...]) + bn1_r[...])
        oh_ref[...] = _dot(t2, wn2_r[...]) + bn2_r[...]

    ws = (w1a, w1b, w1c, b1, w2, b2, wna, wnb, bn1, wn2, bn2)
    return pl.pallas_call(
        body,
        grid=(E_PAD // BE,),
        in_specs=[pl.BlockSpec((BE, nf), lambda i: (i, 0)),
                  pl.BlockSpec((BE, nf), lambda i: (i, 0)),
                  pl.BlockSpec((BE, ef), lambda i: (i, 0))]
                 + [_full_spec(a.shape) for a in ws],
        out_specs=[pl.BlockSpec((BE, eo), lambda i: (i, 0)),
                   pl.BlockSpec((BE, 64), lambda i: (i, 0))],
        out_shape=[jax.ShapeDtypeStruct((E_PAD, eo), jnp.float32),
                   jax.ShapeDtypeStruct((E_PAD, 64), jnp.float32)],
        compiler_params=pltpu.CompilerParams(
            dimension_semantics=("arbitrary",)),
    )(xr, xc, e, *ws)


def _node_call(nf, gout, has_u, x, hs0, hs1, dg0, dg1, batch2, u,
               w21a, w21b, b21, w22, b22, wg1u, wg1g, bg1, wg2, bg2):
    def body(*refs):
        (x_ref, hs0_ref, hs1_ref, dg0_ref, dg1_ref, batch_ref) = refs[:6]
        k = 6
        if has_u:
            u_ref = refs[k]
            k += 1
        (w21a_r, w21b_r, b21_r, w22_r, b22_r) = refs[k:k + 5]
        k += 5
        if has_u:
            wg1u_r = refs[k]
            k += 1
        (wg1g_r, bg1_r, wg2_r, bg2_r) = refs[k:k + 4]
        k += 4
        xn_ref, un_ref = refs[k:k + 2]
        acc = refs[k + 2]
        i = pl.program_id(0)

        @pl.when(i < NBLK)
        def _():
            hsv = hs0_ref[...] + hs1_ref[...]
            degv = dg0_ref[:, 0:1] + dg1_ref[:, 0:1]
            hm = hsv / jnp.maximum(degv, 1.0)
            t = _elu(_dot(x_ref[...], w21a_r[...]) + _dot(hm, w21b_r[...])
                     + b21_r[...])
            xn = _dot(t, w22_r[...]) + b22_r[...]
            xn_ref[...] = xn
            bb = batch_ref[...].reshape(1, BNODE)
            gids = lax.broadcasted_iota(jnp.int32, (G, BNODE), 0)
            oh = (gids == bb).astype(jnp.float32)
            ones = jnp.ones((BNODE, 16), jnp.float32)
            contrib = _dot(oh, jnp.concatenate([xn, ones], axis=1))

            @pl.when(i == 0)
            def _():
                acc[...] = jnp.zeros_like(acc)

            acc[...] += contrib

        @pl.when(i == NBLK)
        def _():
            cnt = acc[:, 32:33]
            gm = acc[:, 0:32] / jnp.maximum(cnt, 1.0)
            z = _dot(gm, wg1g_r[...]) + bg1_r[...]
            if has_u:
                z = z + _dot(u_ref[...], wg1u_r[...])
            tg = _elu(z)
            un_ref[...] = _dot(tg, wg2_r[...]) + bg2_r[...]

    jcap = lambda i: (jnp.minimum(i, NBLK - 1), 0)
    jcap3 = lambda i: (jnp.minimum(i, NBLK - 1), 0)
    in_arrays = [x, hs0, hs1, dg0, dg1, batch2]
    in_specs = [pl.BlockSpec((BNODE, nf), jcap),
                pl.BlockSpec((BNODE, 64), jcap3),
                pl.BlockSpec((BNODE, 64), jcap3),
                pl.BlockSpec((BNODE, 16), jcap3),
                pl.BlockSpec((BNODE, 16), jcap3),
                pl.BlockSpec((BNODE, 1), jcap)]
    if has_u:
        in_arrays.append(u)
        in_specs.append(_full_spec(u.shape))
    ws = [w21a, w21b, b21, w22, b22]
    if has_u:
        ws.append(wg1u)
    ws += [wg1g, bg1, wg2, bg2]
    in_arrays += ws
    in_specs += [_full_spec(a.shape) for a in ws]
    return pl.pallas_call(
        body,
        grid=(NBLK + 1,),
        in_specs=in_specs,
        out_specs=[pl.BlockSpec((BNODE, 32), jcap),
                   pl.BlockSpec((G, gout), lambda i: (0, 0))],
        out_shape=[jax.ShapeDtypeStruct((N, 32), jnp.float32),
                   jax.ShapeDtypeStruct((G, gout), jnp.float32)],
        scratch_shapes=[pltpu.VMEM((G, 48), jnp.float32)],
        compiler_params=pltpu.CompilerParams(
            dimension_semantics=("arbitrary",)),
    )(*in_arrays)


def _final_call(u, w1, b1, w2, b2):
    def body(u_ref, w1_r, b1_r, w2_r, b2_r, o_ref):
        t = _elu(_dot(u_ref[...], w1_r[...]) + b1_r[...])
        o_ref[...] = _dot(t, w2_r[...]) + b2_r[...]

    return pl.pallas_call(
        body,
        out_shape=jax.ShapeDtypeStruct((G, 256), jnp.float32),
    )(u, w1, b1, w2, b2)


def _b2(b):
    return b.reshape(1, -1)


def kernel(x, edge_attr, params, edge_index, batch):
    p = params
    row = edge_index[0]
    col = edge_index[1]
    zpad = jnp.zeros((PAD,), jnp.int32)
    ridx3 = jnp.concatenate([row, zpad]).reshape(TILES, CPT, CHUNK)
    cidx3 = jnp.concatenate([col, zpad]).reshape(TILES, CPT, CHUNK)
    sidx3 = jnp.concatenate(
        [col, jnp.full((PAD,), N, jnp.int32)]).reshape(TILES, CPT, CHUNK)
    e0 = jnp.concatenate(
        [edge_attr, jnp.zeros((PAD, DE), jnp.float32)], axis=0)
    zeros64 = jnp.zeros((ROWS_PT, 64), jnp.float32)
    zeros16 = jnp.zeros((ROWS_PT, 16), jnp.float32)
    ones16 = jnp.ones((CHUNK, 16), jnp.float32)
    batch2 = batch.reshape(N, 1)

    dg0, dg1 = _sc_degree()(sidx3, zeros16, ones16)

    sx, qx = _stats(x, BNODE)
    se, qe = _stats(edge_attr, 8000)
    mx = sx / N
    vx = qx / N - mx * mx
    me = se / E
    ve = qe / E - me * me
    s_x = p["bn_node"]["g"] / jnp.sqrt(vx + 1e-5)
    t_x = p["bn_node"]["b"] - mx * s_x
    s_e = p["bn_edge"]["g"] / jnp.sqrt(ve + 1e-5)
    t_e = p["bn_edge"]["b"] - me * s_e

    m1 = p["m1"]
    e1w, e1b = m1["e1"]["w"], m1["e1"]["b"]
    w1a = e1w[:DN] * s_x[:, None]
    w1b = e1w[DN:2 * DN] * s_x[:, None]
    w1c = e1w[2 * DN:] * s_e[:, None]
    b1 = (e1b + t_x @ e1w[:DN] + t_x @ e1w[DN:2 * DN] + t_e @ e1w[2 * DN:])
    n11w, n11b = m1["n11"]["w"], m1["n11"]["b"]
    wna = n11w[:DN] * s_x[:, None]
    wnb = n11w[DN:]
    bn1 = n11b + t_x @ n11w[:DN]
    n21w, n21b = m1["n21"]["w"], m1["n21"]["b"]
    w21a = n21w[:DN] * s_x[:, None]
    w21b = n21w[DN:]
    b21 = n21b + t_x @ n21w[:DN]

    xr, xc = _sc_gather(DN)(x, ridx3, cidx3)
    ecur, h = _edge_call(
        DN, DE, 256, xr, xc, e0,
        w1a, w1b, w1c, _b2(b1), m1["e2"]["w"], _b2(m1["e2"]["b"]),
        wna, wnb, _b2(bn1), m1["n12"]["w"], _b2(m1["n12"]["b"]))
    hs0, hs1 = _sc_scatter(64)(h, sidx3, zeros64)
    xcur, u = _node_call(
        DN, 32, False, x, hs0, hs1, dg0, dg1, batch2, None,
        w21a, w21b, _b2(b21), m1["n22"]["w"], _b2(m1["n22"]["b"]),
        None, m1["g1"]["w"], _b2(m1["g1"]["b"]),
        m1["g2"]["w"], _b2(m1["g2"]["b"]))

    for name in ("m2", "m3", "m4", "m5", "m6", "m7"):
        mp = p[name]
        eo = mp["e2"]["w"].shape[1]
        gout = mp["g2"]["w"].shape[1]
        e1w = mp["e1"]["w"]
        n11w = mp["n11"]["w"]
        n21w = mp["n21"]["w"]
        g1w = mp["g1"]["w"]
        xr, xc = _sc_gather(32)(xcur, ridx3, cidx3)
        ecur, h = _edge_call(
            32, 256, eo, xr, xc, ecur,
            e1w[:32], e1w[32:64], e1w[64:], _b2(mp["e1"]["b"]),
            mp["e2"]["w"], _b2(mp["e2"]["b"]),
            n11w[:32], n11w[32:], _b2(mp["n11"]["b"]),
            mp["n12"]["w"], _b2(mp["n12"]["b"]))
        hs0, hs1 = _sc_scatter(64)(h, sidx3, zeros64)
        xcur, u = _node_call(
            32, gout, True, xcur, hs0, hs1, dg0, dg1, batch2, u,
            n21w[:32], n21w[32:], _b2(mp["n21"]["b"]),
            mp["n22"]["w"], _b2(mp["n22"]["b"]),
            g1w[:32], g1w[32:], _b2(mp["g1"]["b"]),
            mp["g2"]["w"], _b2(mp["g2"]["b"]))

    return _final_call(u, p["lin1"]["w"], _b2(p["lin1"]["b"]),
                       p["lin2"]["w"], _b2(p["lin2"]["b"]))


# speedup vs baseline: 1.0514x; 1.0514x over previous
import functools

import jax
import jax.numpy as jnp
from jax import lax
from jax.experimental import pallas as pl
from jax.experimental.pallas import tpu as pltpu
from jax.experimental.pallas import tpu_sc as plsc

N = 10000
E = 320000
DN = 128
DE = 16
G = 256

NC, NS = 2, 16
TILES = NC * NS
CHUNK = 128
CPT = 80
EPT = CHUNK * CPT
E_PAD = TILES * EPT
PAD = E_PAD - E
ROWS_PT = 632
N_ACC = ROWS_PT * NS
BE = 2048
BNODE = 1000
NBLK = N // BNODE

@functools.lru_cache(maxsize=None)
def _mesh():
    return plsc.VectorSubcoreMesh(core_axis_name="c", subcore_axis_name="s",
                                  num_cores=NC, num_subcores=NS)


def _elu(v):
    return jnp.where(v > 0, v, jnp.exp(jnp.minimum(v, 0.0)) - 1.0)


def _dot(a, b):
    return jnp.dot(a, b, preferred_element_type=jnp.float32)


def _full_spec(shape):
    n = len(shape)
    return pl.BlockSpec(shape, lambda i, _n=n: (0,) * _n)


@functools.lru_cache(maxsize=None)
def _sc_gather(nf):
    nb = 2 if nf > 64 else 4
    ng = CPT // nb
    @functools.partial(
        pl.kernel,
        out_type=(jax.ShapeDtypeStruct((E_PAD, nf), jnp.bfloat16),
                  jax.ShapeDtypeStruct((E_PAD, nf), jnp.bfloat16)),
        mesh=_mesh(),
        scratch_types=[pltpu.VMEM((CPT, CHUNK), jnp.int32),
                       pltpu.VMEM((CPT, CHUNK), jnp.int32),
                       pltpu.VMEM((nb, CHUNK, nf), jnp.bfloat16),
                       pltpu.VMEM((nb, CHUNK, nf), jnp.bfloat16),
                       pltpu.SemaphoreType.DMA,
                       pltpu.SemaphoreType.DMA],
        compiler_params=pltpu.CompilerParams(use_tc_tiling_on_sc=False),
    )
    def gath(x_hbm, ridx, cidx, xr_out, xc_out, idxr, idxc, bufr, bufc,
             gsem, ssem):
        cid = lax.axis_index("c")
        sid = lax.axis_index("s")
        wid = sid * NC + cid
        base = wid * EPT
        pltpu.sync_copy(ridx.at[wid], idxr)
        pltpu.sync_copy(cidx.at[wid], idxc)

        def group(k, c):
            descs = []
            for b in range(nb):
                j = k * nb + b
                descs.append(
                    pltpu.async_copy(x_hbm.at[idxr.at[j]], bufr.at[b], gsem))
                descs.append(
                    pltpu.async_copy(x_hbm.at[idxc.at[j]], bufc.at[b], gsem))
            for d in descs:
                d.wait()
            descs = []
            for b in range(nb):
                j = k * nb + b
                dst = pl.ds(base + j * CHUNK, CHUNK)
                descs.append(
                    pltpu.async_copy(bufr.at[b], xr_out.at[dst], ssem))
                descs.append(
                    pltpu.async_copy(bufc.at[b], xc_out.at[dst], ssem))
            for d in descs:
                d.wait()
            return c

        lax.fori_loop(0, ng, group, 0)
    return gath


@functools.lru_cache(maxsize=None)
def _sc_scatter(w):
    @functools.partial(
        pl.kernel,
        out_type=(jax.ShapeDtypeStruct((N_ACC, w), jnp.float32),
                  jax.ShapeDtypeStruct((N_ACC, w), jnp.float32)),
        mesh=_mesh(),
        scratch_types=[pltpu.VMEM((CPT, CHUNK), jnp.int32),
                       pltpu.VMEM((8, CHUNK, w), jnp.float32),
                       pltpu.VMEM_SHARED((N_ACC, w), jnp.float32),
                       pltpu.SemaphoreType.DMA,
                       pltpu.SemaphoreType.DMA],
        compiler_params=pltpu.CompilerParams(use_tc_tiling_on_sc=False),
    )
    def scat(h_hbm, idx3, z_hbm, out0, out1, idxv, hbuf, acc, lsem, asem):
        cid = lax.axis_index("c")
        sid = lax.axis_index("s")
        wid = sid * NC + cid
        base = wid * EPT
        sl = pl.ds(sid * ROWS_PT, ROWS_PT)
        pltpu.sync_copy(z_hbm, acc.at[sl])
        plsc.subcore_barrier()
        pltpu.sync_copy(idx3.at[wid], idxv)
        nb = 8

        def group(k, c):
            descs = []
            for b in range(nb):
                j = k * nb + b
                descs.append(pltpu.async_copy(
                    h_hbm.at[pl.ds(base + j * CHUNK, CHUNK)], hbuf.at[b],
                    lsem))
            for d in descs:
                d.wait()
            descs = []
            for b in range(nb):
                j = k * nb + b
                descs.append(pltpu.async_copy(
                    hbuf.at[b], acc.at[idxv.at[j]], asem, add=True))
            for d in descs:
                d.wait()
            return c

        lax.fori_loop(0, CPT // nb, group, 0)
        plsc.subcore_barrier()

        @pl.when(cid == 0)
        def _():
            pltpu.sync_copy(acc.at[sl], out0.at[sl])

        @pl.when(cid == 1)
        def _():
            pltpu.sync_copy(acc.at[sl], out1.at[sl])
    return scat


@functools.lru_cache(maxsize=None)
def _sc_degree():
    @functools.partial(
        pl.kernel,
        out_type=(jax.ShapeDtypeStruct((N_ACC, 16), jnp.float32),
                  jax.ShapeDtypeStruct((N_ACC, 16), jnp.float32)),
        mesh=_mesh(),
        scratch_types=[pltpu.VMEM((CPT, CHUNK), jnp.int32),
                       pltpu.VMEM((CHUNK, 16), jnp.float32),
                       pltpu.VMEM_SHARED((N_ACC, 16), jnp.float32),
                       pltpu.SemaphoreType.DMA],
        compiler_params=pltpu.CompilerParams(use_tc_tiling_on_sc=False),
    )
    def degk(idx3, z_hbm, ones_hbm, out0, out1, idxv, obuf, acc, asem):
        cid = lax.axis_index("c")
        sid = lax.axis_index("s")
        wid = sid * NC + cid
        sl = pl.ds(sid * ROWS_PT, ROWS_PT)
        pltpu.sync_copy(z_hbm, acc.at[sl])
        pltpu.sync_copy(ones_hbm, obuf)
        plsc.subcore_barrier()
        pltpu.sync_copy(idx3.at[wid], idxv)
        nb = 8

        def group(k, c):
            descs = [pltpu.async_copy(obuf, acc.at[idxv.at[k * nb + b]],
                                      asem, add=True) for b in range(nb)]
            for d in descs:
                d.wait()
            return c

        lax.fori_loop(0, CPT // nb, group, 0)
        plsc.subcore_barrier()

        @pl.when(cid == 0)
        def _():
            pltpu.sync_copy(acc.at[sl], out0.at[sl])

        @pl.when(cid == 1)
        def _():
            pltpu.sync_copy(acc.at[sl], out1.at[sl])
    return degk


def _stats(x, bs, want_cast=False):
    m, c = x.shape
    nb = m // bs

    def body(x_ref, o_ref, *rest):
        i = pl.program_id(0)

        @pl.when(i == 0)
        def _():
            o_ref[...] = jnp.zeros_like(o_ref)

        xb = x_ref[...]
        o_ref[0:1, :] += jnp.sum(xb, axis=0, keepdims=True)
        o_ref[1:2, :] += jnp.sum(xb * xb, axis=0, keepdims=True)
        if want_cast:
            rest[0][...] = xb.astype(jnp.bfloat16)

    out_specs = [pl.BlockSpec((8, c), lambda i: (0, 0))]
    out_shape = [jax.ShapeDtypeStruct((8, c), jnp.float32)]
    if want_cast:
        out_specs.append(pl.BlockSpec((bs, c), lambda i: (i, 0)))
        out_shape.append(jax.ShapeDtypeStruct((m, c), jnp.bfloat16))
    out = pl.pallas_call(
        body,
        grid=(nb,),
        in_specs=[pl.BlockSpec((bs, c), lambda i: (i, 0))],
        out_specs=out_specs,
        out_shape=out_shape,
    )(x)
    if want_cast:
        return out[0][0], out[0][1], out[1]
    return out[0][0], out[0][1]


def _edge_call(nf, ef, eo, xr, xc, e, w1a, w1b, w1c, b1, w2, b2,
               wna, wnb, bn1, wn2, bn2):
    def body(xr_ref, xc_ref, e_ref, w1a_r, w1b_r, w1c_r, b1_r, w2_r, b2_r,
             wna_r, wnb_r, bn1_r, wn2_r, bn2_r, oe_ref, oh_ref):
        xrv = xr_ref[...].astype(jnp.float32)
        xcv = xc_ref[...].astype(jnp.float32)
        ev = e_ref[...]
        t1 = _elu(_dot(xrv, w1a_r[...]) + _dot(xcv, w1b_r[...])
                  + _dot(ev, w1c_r[...]) + b1_r[...])
        en = _dot(t1, w2_r[...]) + b2_r[...]
        oe_ref[...] = en
        t2 = _elu(_dot(xrv, wna_r[...]) + _dot(en, wnb_r[...]) + bn1_r[...])
        oh_ref[...] = _dot(t2, wn2_r[...]) + bn2_r[...]

    ws = (w1a, w1b, w1c, b1, w2, b2, wna, wnb, bn1, wn2, bn2)
    return pl.pallas_call(
        body,
        grid=(E_PAD // BE,),
        in_specs=[pl.BlockSpec((BE, nf), lambda i: (i, 0)),
                  pl.BlockSpec((BE, nf), lambda i: (i, 0)),
                  pl.BlockSpec((BE, ef), lambda i: (i, 0))]
                 + [_full_spec(a.shape) for a in ws],
        out_specs=[pl.BlockSpec((BE, eo), lambda i: (i, 0)),
                   pl.BlockSpec((BE, 64), lambda i: (i, 0))],
        out_shape=[jax.ShapeDtypeStruct((E_PAD, eo), jnp.float32),
                   jax.ShapeDtypeStruct((E_PAD, 64), jnp.float32)],
        compiler_params=pltpu.CompilerParams(
            dimension_semantics=("arbitrary",)),
    )(xr, xc, e, *ws)


def _node_call(nf, gout, has_u, x, hs0, hs1, dg0, dg1, batch2, u,
               w21a, w21b, b21, w22, b22, wg1u, wg1g, bg1, wg2, bg2):
    def body(*refs):
        (x_ref, hs0_ref, hs1_ref, dg0_ref, dg1_ref, batch_ref) = refs[:6]
        k = 6
        if has_u:
            u_ref = refs[k]
            k += 1
        (w21a_r, w21b_r, b21_r, w22_r, b22_r) = refs[k:k + 5]
        k += 5
        if has_u:
            wg1u_r = refs[k]
            k += 1
        (wg1g_r, bg1_r, wg2_r, bg2_r) = refs[k:k + 4]
        k += 4
        xn_ref, un_ref = refs[k:k + 2]
        acc = refs[k + 2]
        i = pl.program_id(0)

        @pl.when(i < NBLK)
        def _():
            hsv = hs0_ref[...] + hs1_ref[...]
            degv = dg0_ref[:, 0:1] + dg1_ref[:, 0:1]
            hm = hsv / jnp.maximum(degv, 1.0)
            xv = x_ref[...].astype(jnp.float32)
            t = _elu(_dot(xv, w21a_r[...]) + _dot(hm, w21b_r[...])
                     + b21_r[...])
            xn = _dot(t, w22_r[...]) + b22_r[...]
            xn_ref[...] = xn.astype(jnp.bfloat16)
            bb = batch_ref[...].reshape(1, BNODE)
            gids = lax.broadcasted_iota(jnp.int32, (G, BNODE), 0)
            oh = (gids == bb).astype(jnp.float32)
            ones = jnp.ones((BNODE, 16), jnp.float32)
            contrib = _dot(oh, jnp.concatenate([xn, ones], axis=1))

            @pl.when(i == 0)
            def _():
                acc[...] = jnp.zeros_like(acc)

            acc[...] += contrib

        @pl.when(i == NBLK)
        def _():
            cnt = acc[:, 32:33]
            gm = acc[:, 0:32] / jnp.maximum(cnt, 1.0)
            z = _dot(gm, wg1g_r[...]) + bg1_r[...]
            if has_u:
                z = z + _dot(u_ref[...], wg1u_r[...])
            tg = _elu(z)
            un_ref[...] = _dot(tg, wg2_r[...]) + bg2_r[...]

    jcap = lambda i: (jnp.minimum(i, NBLK - 1), 0)
    jcap3 = lambda i: (jnp.minimum(i, NBLK - 1), 0)
    in_arrays = [x, hs0, hs1, dg0, dg1, batch2]
    in_specs = [pl.BlockSpec((BNODE, nf), jcap),
                pl.BlockSpec((BNODE, 64), jcap3),
                pl.BlockSpec((BNODE, 64), jcap3),
                pl.BlockSpec((BNODE, 16), jcap3),
                pl.BlockSpec((BNODE, 16), jcap3),
                pl.BlockSpec((BNODE, 1), jcap)]
    if has_u:
        in_arrays.append(u)
        in_specs.append(_full_spec(u.shape))
    ws = [w21a, w21b, b21, w22, b22]
    if has_u:
        ws.append(wg1u)
    ws += [wg1g, bg1, wg2, bg2]
    in_arrays += ws
    in_specs += [_full_spec(a.shape) for a in ws]
    return pl.pallas_call(
        body,
        grid=(NBLK + 1,),
        in_specs=in_specs,
        out_specs=[pl.BlockSpec((BNODE, 32), jcap),
                   pl.BlockSpec((G, gout), lambda i: (0, 0))],
        out_shape=[jax.ShapeDtypeStruct((N, 32), jnp.bfloat16),
                   jax.ShapeDtypeStruct((G, gout), jnp.float32)],
        scratch_shapes=[pltpu.VMEM((G, 48), jnp.float32)],
        compiler_params=pltpu.CompilerParams(
            dimension_semantics=("arbitrary",)),
    )(*in_arrays)


def _final_call(u, w1, b1, w2, b2):
    def body(u_ref, w1_r, b1_r, w2_r, b2_r, o_ref):
        t = _elu(_dot(u_ref[...], w1_r[...]) + b1_r[...])
        o_ref[...] = _dot(t, w2_r[...]) + b2_r[...]

    return pl.pallas_call(
        body,
        out_shape=jax.ShapeDtypeStruct((G, 256), jnp.float32),
    )(u, w1, b1, w2, b2)


def _b2(b):
    return b.reshape(1, -1)


def kernel(x, edge_attr, params, edge_index, batch):
    p = params
    row = edge_index[0]
    col = edge_index[1]
    zpad = jnp.zeros((PAD,), jnp.int32)
    ridx3 = jnp.concatenate([row, zpad]).reshape(TILES, CPT, CHUNK)
    cidx3 = jnp.concatenate([col, zpad]).reshape(TILES, CPT, CHUNK)
    sidx3 = jnp.concatenate(
        [col, jnp.full((PAD,), N, jnp.int32)]).reshape(TILES, CPT, CHUNK)
    e0 = jnp.concatenate(
        [edge_attr, jnp.zeros((PAD, DE), jnp.float32)], axis=0)
    zeros64 = jnp.zeros((ROWS_PT, 64), jnp.float32)
    zeros16 = jnp.zeros((ROWS_PT, 16), jnp.float32)
    ones16 = jnp.ones((CHUNK, 16), jnp.float32)
    batch2 = batch.reshape(N, 1)

    dg0, dg1 = _sc_degree()(sidx3, zeros16, ones16)

    sx, qx, x16 = _stats(x, BNODE, want_cast=True)
    se, qe = _stats(edge_attr, 8000)
    mx = sx / N
    vx = qx / N - mx * mx
    me = se / E
    ve = qe / E - me * me
    s_x = p["bn_node"]["g"] / jnp.sqrt(vx + 1e-5)
    t_x = p["bn_node"]["b"] - mx * s_x
    s_e = p["bn_edge"]["g"] / jnp.sqrt(ve + 1e-5)
    t_e = p["bn_edge"]["b"] - me * s_e

    m1 = p["m1"]
    e1w, e1b = m1["e1"]["w"], m1["e1"]["b"]
    w1a = e1w[:DN] * s_x[:, None]
    w1b = e1w[DN:2 * DN] * s_x[:, None]
    w1c = e1w[2 * DN:] * s_e[:, None]
    b1 = (e1b + t_x @ e1w[:DN] + t_x @ e1w[DN:2 * DN] + t_e @ e1w[2 * DN:])
    n11w, n11b = m1["n11"]["w"], m1["n11"]["b"]
    wna = n11w[:DN] * s_x[:, None]
    wnb = n11w[DN:]
    bn1 = n11b + t_x @ n11w[:DN]
    n21w, n21b = m1["n21"]["w"], m1["n21"]["b"]
    w21a = n21w[:DN] * s_x[:, None]
    w21b = n21w[DN:]
    b21 = n21b + t_x @ n21w[:DN]

    xr, xc = _sc_gather(DN)(x16, ridx3, cidx3)
    ecur, h = _edge_call(
        DN, DE, 256, xr, xc, e0,
        w1a, w1b, w1c, _b2(b1), m1["e2"]["w"], _b2(m1["e2"]["b"]),
        wna, wnb, _b2(bn1), m1["n12"]["w"], _b2(m1["n12"]["b"]))
    hs0, hs1 = _sc_scatter(64)(h, sidx3, zeros64)
    xcur, u = _node_call(
        DN, 32, False, x, hs0, hs1, dg0, dg1, batch2, None,
        w21a, w21b, _b2(b21), m1["n22"]["w"], _b2(m1["n22"]["b"]),
        None, m1["g1"]["w"], _b2(m1["g1"]["b"]),
        m1["g2"]["w"], _b2(m1["g2"]["b"]))

    for name in ("m2", "m3", "m4", "m5", "m6", "m7"):
        mp = p[name]
        eo = mp["e2"]["w"].shape[1]
        gout = mp["g2"]["w"].shape[1]
        e1w = mp["e1"]["w"]
        n11w = mp["n11"]["w"]
        n21w = mp["n21"]["w"]
        g1w = mp["g1"]["w"]
        xr, xc = _sc_gather(32)(xcur, ridx3, cidx3)
        ecur, h = _edge_call(
            32, 256, eo, xr, xc, ecur,
            e1w[:32], e1w[32:64], e1w[64:], _b2(mp["e1"]["b"]),
            mp["e2"]["w"], _b2(mp["e2"]["b"]),
            n11w[:32], n11w[32:], _b2(mp["n11"]["b"]),
            mp["n12"]["w"], _b2(mp["n12"]["b"]))
        hs0, hs1 = _sc_scatter(64)(h, sidx3, zeros64)
        xcur, u = _node_call(
            32, gout, True, xcur, hs0, hs1, dg0, dg1, batch2, u,
            n21w[:32], n21w[32:], _b2(mp["n21"]["b"]),
            mp["n22"]["w"], _b2(mp["n22"]["b"]),
            g1w[:32], g1w[32:], _b2(mp["g1"]["b"]),
            mp["g2"]["w"], _b2(mp["g2"]["b"]))

    return _final_call(u, p["lin1"]["w"], _b2(p["lin1"]["b"]),
                       p["lin2"]["w"], _b2(p["lin2"]["b"]))


# speedup vs baseline: 1.1238x; 1.0688x over previous
import functools

import jax
import jax.numpy as jnp
from jax import lax
from jax.experimental import pallas as pl
from jax.experimental.pallas import tpu as pltpu
from jax.experimental.pallas import tpu_sc as plsc

N = 10000
E = 320000
DN = 128
DE = 16
G = 256

NC, NS = 2, 16
TILES = NC * NS
CHUNK = 128
CPT = 80
EPT = CHUNK * CPT
E_PAD = TILES * EPT
PAD = E_PAD - E
ROWS_PT = 632
N_ACC = ROWS_PT * NS
BE = 2048
BNODE = 1000
NBLK = N // BNODE

@functools.lru_cache(maxsize=None)
def _mesh():
    return plsc.VectorSubcoreMesh(core_axis_name="c", subcore_axis_name="s",
                                  num_cores=NC, num_subcores=NS)


def _elu(v):
    return jnp.where(v > 0, v, jnp.exp(jnp.minimum(v, 0.0)) - 1.0)


def _dot(a, b):
    return jnp.dot(a, b, preferred_element_type=jnp.float32)


def _full_spec(shape):
    n = len(shape)
    return pl.BlockSpec(shape, lambda i, _n=n: (0,) * _n)


@functools.lru_cache(maxsize=None)
def _sc_gather(nf):
    nb = 2 if nf > 64 else 4
    ng = CPT // nb
    @functools.partial(
        pl.kernel,
        out_type=(jax.ShapeDtypeStruct((E_PAD, nf), jnp.bfloat16),
                  jax.ShapeDtypeStruct((E_PAD, nf), jnp.bfloat16)),
        mesh=_mesh(),
        scratch_types=[pltpu.VMEM((CPT, CHUNK), jnp.int32),
                       pltpu.VMEM((CPT, CHUNK), jnp.int32),
                       pltpu.VMEM((nb, CHUNK, nf), jnp.bfloat16),
                       pltpu.VMEM((nb, CHUNK, nf), jnp.bfloat16),
                       pltpu.SemaphoreType.DMA,
                       pltpu.SemaphoreType.DMA],
        compiler_params=pltpu.CompilerParams(use_tc_tiling_on_sc=False),
    )
    def gath(x_hbm, ridx, cidx, xr_out, xc_out, idxr, idxc, bufr, bufc,
             gsem, ssem):
        cid = lax.axis_index("c")
        sid = lax.axis_index("s")
        wid = sid * NC + cid
        base = wid * EPT
        pltpu.sync_copy(ridx.at[wid], idxr)
        pltpu.sync_copy(cidx.at[wid], idxc)

        def group(k, c):
            descs = []
            for b in range(nb):
                j = k * nb + b
                descs.append(
                    pltpu.async_copy(x_hbm.at[idxr.at[j]], bufr.at[b], gsem))
                descs.append(
                    pltpu.async_copy(x_hbm.at[idxc.at[j]], bufc.at[b], gsem))
            for d in descs:
                d.wait()
            descs = []
            for b in range(nb):
                j = k * nb + b
                dst = pl.ds(base + j * CHUNK, CHUNK)
                descs.append(
                    pltpu.async_copy(bufr.at[b], xr_out.at[dst], ssem))
                descs.append(
                    pltpu.async_copy(bufc.at[b], xc_out.at[dst], ssem))
            for d in descs:
                d.wait()
            return c

        lax.fori_loop(0, ng, group, 0)
    return gath


@functools.lru_cache(maxsize=None)
def _sc_scatter(w):
    @functools.partial(
        pl.kernel,
        out_type=(jax.ShapeDtypeStruct((N_ACC, w), jnp.float32),
                  jax.ShapeDtypeStruct((N_ACC, w), jnp.float32)),
        mesh=_mesh(),
        scratch_types=[pltpu.VMEM((CPT, CHUNK), jnp.int32),
                       pltpu.VMEM((8, CHUNK, w), jnp.float32),
                       pltpu.VMEM_SHARED((N_ACC, w), jnp.float32),
                       pltpu.SemaphoreType.DMA,
                       pltpu.SemaphoreType.DMA],
        compiler_params=pltpu.CompilerParams(use_tc_tiling_on_sc=False),
    )
    def scat(h_hbm, idx3, z_hbm, out0, out1, idxv, hbuf, acc, lsem, asem):
        cid = lax.axis_index("c")
        sid = lax.axis_index("s")
        wid = sid * NC + cid
        base = wid * EPT
        sl = pl.ds(sid * ROWS_PT, ROWS_PT)
        pltpu.sync_copy(z_hbm, acc.at[sl])
        plsc.subcore_barrier()
        pltpu.sync_copy(idx3.at[wid], idxv)
        nb = 8

        def group(k, c):
            descs = []
            for b in range(nb):
                j = k * nb + b
                descs.append(pltpu.async_copy(
                    h_hbm.at[pl.ds(base + j * CHUNK, CHUNK)], hbuf.at[b],
                    lsem))
            for d in descs:
                d.wait()
            descs = []
            for b in range(nb):
                j = k * nb + b
                descs.append(pltpu.async_copy(
                    hbuf.at[b], acc.at[idxv.at[j]], asem, add=True))
            for d in descs:
                d.wait()
            return c

        lax.fori_loop(0, CPT // nb, group, 0)
        plsc.subcore_barrier()

        @pl.when(cid == 0)
        def _():
            pltpu.sync_copy(acc.at[sl], out0.at[sl])

        @pl.when(cid == 1)
        def _():
            pltpu.sync_copy(acc.at[sl], out1.at[sl])
    return scat


@functools.lru_cache(maxsize=None)
def _sc_degree():
    @functools.partial(
        pl.kernel,
        out_type=(jax.ShapeDtypeStruct((N_ACC, 16), jnp.float32),
                  jax.ShapeDtypeStruct((N_ACC, 16), jnp.float32)),
        mesh=_mesh(),
        scratch_types=[pltpu.VMEM((CPT, CHUNK), jnp.int32),
                       pltpu.VMEM((CHUNK, 16), jnp.float32),
                       pltpu.VMEM_SHARED((N_ACC, 16), jnp.float32),
                       pltpu.SemaphoreType.DMA],
        compiler_params=pltpu.CompilerParams(use_tc_tiling_on_sc=False),
    )
    def degk(idx3, z_hbm, ones_hbm, out0, out1, idxv, obuf, acc, asem):
        cid = lax.axis_index("c")
        sid = lax.axis_index("s")
        wid = sid * NC + cid
        sl = pl.ds(sid * ROWS_PT, ROWS_PT)
        pltpu.sync_copy(z_hbm, acc.at[sl])
        pltpu.sync_copy(ones_hbm, obuf)
        plsc.subcore_barrier()
        pltpu.sync_copy(idx3.at[wid], idxv)
        nb = 8

        def group(k, c):
            descs = [pltpu.async_copy(obuf, acc.at[idxv.at[k * nb + b]],
                                      asem, add=True) for b in range(nb)]
            for d in descs:
                d.wait()
            return c

        lax.fori_loop(0, CPT // nb, group, 0)
        plsc.subcore_barrier()

        @pl.when(cid == 0)
        def _():
            pltpu.sync_copy(acc.at[sl], out0.at[sl])

        @pl.when(cid == 1)
        def _():
            pltpu.sync_copy(acc.at[sl], out1.at[sl])
    return degk


def _stats(x, bs, want_cast=False):
    m, c = x.shape
    nb = m // bs

    def body(x_ref, o_ref, *rest):
        i = pl.program_id(0)

        @pl.when(i == 0)
        def _():
            o_ref[...] = jnp.zeros_like(o_ref)

        xb = x_ref[...]
        o_ref[0:1, :] += jnp.sum(xb, axis=0, keepdims=True)
        o_ref[1:2, :] += jnp.sum(xb * xb, axis=0, keepdims=True)
        if want_cast:
            rest[0][...] = xb.astype(jnp.bfloat16)

    out_specs = [pl.BlockSpec((8, c), lambda i: (0, 0))]
    out_shape = [jax.ShapeDtypeStruct((8, c), jnp.float32)]
    if want_cast:
        out_specs.append(pl.BlockSpec((bs, c), lambda i: (i, 0)))
        out_shape.append(jax.ShapeDtypeStruct((m, c), jnp.bfloat16))
    out = pl.pallas_call(
        body,
        grid=(nb,),
        in_specs=[pl.BlockSpec((bs, c), lambda i: (i, 0))],
        out_specs=out_specs,
        out_shape=out_shape,
    )(x)
    if want_cast:
        return out[0][0], out[0][1], out[1]
    return out[0][0], out[0][1]


def _edge_call(nf, ef, xr, xc, e, w1a, w1b, w1c, b1, w2, b2,
               wna, wnb, bn1, wn2, bn2, wnext):
    def body(*refs):
        (xr_ref, xc_ref, e_ref) = refs[:3]
        k = 3
        if w1c is not None:
            w1c_r = refs[k]
            k += 1
        (w1a_r, w1b_r, b1_r, w2_r, b2_r,
         wna_r, wnb_r, bn1_r, wn2_r, bn2_r) = refs[k:k + 10]
        k += 10
        if wnext is not None:
            wnext_r = refs[k]
            k += 1
        oh_ref = refs[k]
        xrv = xr_ref[...].astype(jnp.float32)
        xcv = xc_ref[...].astype(jnp.float32)
        ev = e_ref[...]
        z = _dot(xrv, w1a_r[...]) + _dot(xcv, w1b_r[...]) + b1_r[...]
        if w1c is not None:
            z = z + _dot(ev, w1c_r[...])
        else:
            z = z + ev
        t1 = _elu(z)
        en = _dot(t1, w2_r[...]) + b2_r[...]
        if wnext is not None:
            refs[k + 1][...] = _dot(en, wnext_r[...])
        t2 = _elu(_dot(xrv, wna_r[...]) + _dot(en, wnb_r[...]) + bn1_r[...])
        oh_ref[...] = _dot(t2, wn2_r[...]) + bn2_r[...]

    ws = []
    if w1c is not None:
        ws.append(w1c)
    ws += [w1a, w1b, b1, w2, b2, wna, wnb, bn1, wn2, bn2]
    if wnext is not None:
        ws.append(wnext)
    out_specs = [pl.BlockSpec((BE, 64), lambda i: (i, 0))]
    out_shape = [jax.ShapeDtypeStruct((E_PAD, 64), jnp.float32)]
    if wnext is not None:
        out_specs.append(pl.BlockSpec((BE, 32), lambda i: (i, 0)))
        out_shape.append(jax.ShapeDtypeStruct((E_PAD, 32), jnp.float32))
    out = pl.pallas_call(
        body,
        grid=(E_PAD // BE,),
        in_specs=[pl.BlockSpec((BE, nf), lambda i: (i, 0)),
                  pl.BlockSpec((BE, nf), lambda i: (i, 0)),
                  pl.BlockSpec((BE, ef), lambda i: (i, 0))]
                 + [_full_spec(a.shape) for a in ws],
        out_specs=out_specs,
        out_shape=out_shape,
        compiler_params=pltpu.CompilerParams(
            dimension_semantics=("arbitrary",)),
    )(xr, xc, e, *ws)
    if wnext is not None:
        return out[1], out[0]
    return None, out[0]


def _node_call(nf, gout, has_u, x, hs0, hs1, dg0, dg1, batch2, u,
               w21a, w21b, b21, w22, b22, wg1u, wg1g, bg1, wg2, bg2):
    def body(*refs):
        (x_ref, hs0_ref, hs1_ref, dg0_ref, dg1_ref, batch_ref) = refs[:6]
        k = 6
        if has_u:
            u_ref = refs[k]
            k += 1
        (w21a_r, w21b_r, b21_r, w22_r, b22_r) = refs[k:k + 5]
        k += 5
        if has_u:
            wg1u_r = refs[k]
            k += 1
        (wg1g_r, bg1_r, wg2_r, bg2_r) = refs[k:k + 4]
        k += 4
        xn_ref, un_ref = refs[k:k + 2]
        acc = refs[k + 2]
        i = pl.program_id(0)

        @pl.when(i < NBLK)
        def _():
            hsv = hs0_ref[...] + hs1_ref[...]
            degv = dg0_ref[:, 0:1] + dg1_ref[:, 0:1]
            hm = hsv / jnp.maximum(degv, 1.0)
            xv = x_ref[...].astype(jnp.float32)
            t = _elu(_dot(xv, w21a_r[...]) + _dot(hm, w21b_r[...])
                     + b21_r[...])
            xn = _dot(t, w22_r[...]) + b22_r[...]
            xn_ref[...] = xn.astype(jnp.bfloat16)
            bb = batch_ref[...].reshape(1, BNODE)
            gids = lax.broadcasted_iota(jnp.int32, (G, BNODE), 0)
            oh = (gids == bb).astype(jnp.float32)
            ones = jnp.ones((BNODE, 16), jnp.float32)
            contrib = _dot(oh, jnp.concatenate([xn, ones], axis=1))

            @pl.when(i == 0)
            def _():
                acc[...] = jnp.zeros_like(acc)

            acc[...] += contrib

        @pl.when(i == NBLK)
        def _():
            cnt = acc[:, 32:33]
            gm = acc[:, 0:32] / jnp.maximum(cnt, 1.0)
            z = _dot(gm, wg1g_r[...]) + bg1_r[...]
            if has_u:
                z = z + _dot(u_ref[...], wg1u_r[...])
            tg = _elu(z)
            un_ref[...] = _dot(tg, wg2_r[...]) + bg2_r[...]

    jcap = lambda i: (jnp.minimum(i, NBLK - 1), 0)
    jcap3 = lambda i: (jnp.minimum(i, NBLK - 1), 0)
    in_arrays = [x, hs0, hs1, dg0, dg1, batch2]
    in_specs = [pl.BlockSpec((BNODE, nf), jcap),
                pl.BlockSpec((BNODE, 64), jcap3),
                pl.BlockSpec((BNODE, 64), jcap3),
                pl.BlockSpec((BNODE, 16), jcap3),
                pl.BlockSpec((BNODE, 16), jcap3),
                pl.BlockSpec((BNODE, 1), jcap)]
    if has_u:
        in_arrays.append(u)
        in_specs.append(_full_spec(u.shape))
    ws = [w21a, w21b, b21, w22, b22]
    if has_u:
        ws.append(wg1u)
    ws += [wg1g, bg1, wg2, bg2]
    in_arrays += ws
    in_specs += [_full_spec(a.shape) for a in ws]
    return pl.pallas_call(
        body,
        grid=(NBLK + 1,),
        in_specs=in_specs,
        out_specs=[pl.BlockSpec((BNODE, 32), jcap),
                   pl.BlockSpec((G, gout), lambda i: (0, 0))],
        out_shape=[jax.ShapeDtypeStruct((N, 32), jnp.bfloat16),
                   jax.ShapeDtypeStruct((G, gout), jnp.float32)],
        scratch_shapes=[pltpu.VMEM((G, 48), jnp.float32)],
        compiler_params=pltpu.CompilerParams(
            dimension_semantics=("arbitrary",)),
    )(*in_arrays)


def _final_call(u, w1, b1, w2, b2):
    def body(u_ref, w1_r, b1_r, w2_r, b2_r, o_ref):
        t = _elu(_dot(u_ref[...], w1_r[...]) + b1_r[...])
        o_ref[...] = _dot(t, w2_r[...]) + b2_r[...]

    return pl.pallas_call(
        body,
        out_shape=jax.ShapeDtypeStruct((G, 256), jnp.float32),
    )(u, w1, b1, w2, b2)


def _b2(b):
    return b.reshape(1, -1)


def kernel(x, edge_attr, params, edge_index, batch):
    p = params
    row = edge_index[0]
    col = edge_index[1]
    zpad = jnp.zeros((PAD,), jnp.int32)
    ridx3 = jnp.concatenate([row, zpad]).reshape(TILES, CPT, CHUNK)
    cidx3 = jnp.concatenate([col, zpad]).reshape(TILES, CPT, CHUNK)
    sidx3 = jnp.concatenate(
        [col, jnp.full((PAD,), N, jnp.int32)]).reshape(TILES, CPT, CHUNK)
    e0 = jnp.concatenate(
        [edge_attr, jnp.zeros((PAD, DE), jnp.float32)], axis=0)
    zeros64 = jnp.zeros((ROWS_PT, 64), jnp.float32)
    zeros16 = jnp.zeros((ROWS_PT, 16), jnp.float32)
    ones16 = jnp.ones((CHUNK, 16), jnp.float32)
    batch2 = batch.reshape(N, 1)

    dg0, dg1 = _sc_degree()(sidx3, zeros16, ones16)

    sx, qx, x16 = _stats(x, BNODE, want_cast=True)
    se, qe = _stats(edge_attr, 8000)
    mx = sx / N
    vx = qx / N - mx * mx
    me = se / E
    ve = qe / E - me * me
    s_x = p["bn_node"]["g"] / jnp.sqrt(vx + 1e-5)
    t_x = p["bn_node"]["b"] - mx * s_x
    s_e = p["bn_edge"]["g"] / jnp.sqrt(ve + 1e-5)
    t_e = p["bn_edge"]["b"] - me * s_e

    m1 = p["m1"]
    e1w, e1b = m1["e1"]["w"], m1["e1"]["b"]
    w1a = e1w[:DN] * s_x[:, None]
    w1b = e1w[DN:2 * DN] * s_x[:, None]
    w1c = e1w[2 * DN:] * s_e[:, None]
    b1 = (e1b + t_x @ e1w[:DN] + t_x @ e1w[DN:2 * DN] + t_e @ e1w[2 * DN:])
    n11w, n11b = m1["n11"]["w"], m1["n11"]["b"]
    wna = n11w[:DN] * s_x[:, None]
    wnb = n11w[DN:]
    bn1 = n11b + t_x @ n11w[:DN]
    n21w, n21b = m1["n21"]["w"], m1["n21"]["b"]
    w21a = n21w[:DN] * s_x[:, None]
    w21b = n21w[DN:]
    b21 = n21b + t_x @ n21w[:DN]

    xr, xc = _sc_gather(DN)(x16, ridx3, cidx3)
    ecur, h = _edge_call(
        DN, DE, xr, xc, e0,
        w1a, w1b, w1c, _b2(b1), m1["e2"]["w"], _b2(m1["e2"]["b"]),
        wna, wnb, _b2(bn1), m1["n12"]["w"], _b2(m1["n12"]["b"]),
        p["m2"]["e1"]["w"][64:])
    hs0, hs1 = _sc_scatter(64)(h, sidx3, zeros64)
    xcur, u = _node_call(
        DN, 32, False, x, hs0, hs1, dg0, dg1, batch2, None,
        w21a, w21b, _b2(b21), m1["n22"]["w"], _b2(m1["n22"]["b"]),
        None, m1["g1"]["w"], _b2(m1["g1"]["b"]),
        m1["g2"]["w"], _b2(m1["g2"]["b"]))

    names = ("m2", "m3", "m4", "m5", "m6", "m7")
    for li, name in enumerate(names):
        mp = p[name]
        gout = mp["g2"]["w"].shape[1]
        e1w = mp["e1"]["w"]
        n11w = mp["n11"]["w"]
        n21w = mp["n21"]["w"]
        g1w = mp["g1"]["w"]
        wnext = (p[names[li + 1]]["e1"]["w"][64:]
                 if li + 1 < len(names) else None)
        xr, xc = _sc_gather(32)(xcur, ridx3, cidx3)
        ecur, h = _edge_call(
            32, 32, xr, xc, ecur,
            e1w[:32], e1w[32:64], None, _b2(mp["e1"]["b"]),
            mp["e2"]["w"], _b2(mp["e2"]["b"]),
            n11w[:32], n11w[32:], _b2(mp["n11"]["b"]),
            mp["n12"]["w"], _b2(mp["n12"]["b"]), wnext)
        hs0, hs1 = _sc_scatter(64)(h, sidx3, zeros64)
        xcur, u = _node_call(
            32, gout, True, xcur, hs0, hs1, dg0, dg1, batch2, u,
            n21w[:32], n21w[32:], _b2(mp["n21"]["b"]),
            mp["n22"]["w"], _b2(mp["n22"]["b"]),
            g1w[:32], g1w[32:], _b2(mp["g1"]["b"]),
            mp["g2"]["w"], _b2(mp["g2"]["b"]))

    return _final_call(u, p["lin1"]["w"], _b2(p["lin1"]["b"]),
                       p["lin2"]["w"], _b2(p["lin2"]["b"]))


# speedup vs baseline: 1.1454x; 1.0192x over previous
import functools

import jax
import jax.numpy as jnp
from jax import lax
from jax.experimental import pallas as pl
from jax.experimental.pallas import tpu as pltpu
from jax.experimental.pallas import tpu_sc as plsc

N = 10000
E = 320000
DN = 128
DE = 16
G = 256

NC, NS = 2, 16
TILES = NC * NS
CHUNK = 128
CPT = 80
EPT = CHUNK * CPT
E_PAD = TILES * EPT
PAD = E_PAD - E
ROWS_PT = 632
N_ACC = ROWS_PT * NS
BE = 2048
BNODE = 1000
NBLK = N // BNODE

@functools.lru_cache(maxsize=None)
def _mesh():
    return plsc.VectorSubcoreMesh(core_axis_name="c", subcore_axis_name="s",
                                  num_cores=NC, num_subcores=NS)


def _elu(v):
    return jnp.where(v > 0, v, jnp.exp(jnp.minimum(v, 0.0)) - 1.0)


def _dot(a, b):
    return jnp.dot(a, b, preferred_element_type=jnp.float32)


def _full_spec(shape):
    n = len(shape)
    return pl.BlockSpec(shape, lambda i, _n=n: (0,) * _n)


@functools.lru_cache(maxsize=None)
def _sc_gather(nf):
    nb = 2 if nf > 64 else 4
    ng = CPT // nb
    @functools.partial(
        pl.kernel,
        out_type=(jax.ShapeDtypeStruct((E_PAD, nf), jnp.bfloat16),
                  jax.ShapeDtypeStruct((E_PAD, nf), jnp.bfloat16)),
        mesh=_mesh(),
        scratch_types=[pltpu.VMEM((CPT, CHUNK), jnp.int32),
                       pltpu.VMEM((CPT, CHUNK), jnp.int32),
                       pltpu.VMEM((nb, CHUNK, nf), jnp.bfloat16),
                       pltpu.VMEM((nb, CHUNK, nf), jnp.bfloat16),
                       pltpu.SemaphoreType.DMA,
                       pltpu.SemaphoreType.DMA],
        compiler_params=pltpu.CompilerParams(use_tc_tiling_on_sc=False),
    )
    def gath(x_hbm, ridx, cidx, xr_out, xc_out, idxr, idxc, bufr, bufc,
             gsem, ssem):
        cid = lax.axis_index("c")
        sid = lax.axis_index("s")
        wid = sid * NC + cid
        base = wid * EPT
        pltpu.sync_copy(ridx.at[wid], idxr)
        pltpu.sync_copy(cidx.at[wid], idxc)

        def group(k, c):
            descs = []
            for b in range(nb):
                j = k * nb + b
                descs.append(
                    pltpu.async_copy(x_hbm.at[idxr.at[j]], bufr.at[b], gsem))
                descs.append(
                    pltpu.async_copy(x_hbm.at[idxc.at[j]], bufc.at[b], gsem))
            for d in descs:
                d.wait()
            descs = []
            for b in range(nb):
                j = k * nb + b
                dst = pl.ds(base + j * CHUNK, CHUNK)
                descs.append(
                    pltpu.async_copy(bufr.at[b], xr_out.at[dst], ssem))
                descs.append(
                    pltpu.async_copy(bufc.at[b], xc_out.at[dst], ssem))
            for d in descs:
                d.wait()
            return c

        lax.fori_loop(0, ng, group, 0)
    return gath


@functools.lru_cache(maxsize=None)
def _sc_scatter(w):
    @functools.partial(
        pl.kernel,
        out_type=(jax.ShapeDtypeStruct((N_ACC, w), jnp.float32),
                  jax.ShapeDtypeStruct((N_ACC, w), jnp.float32)),
        mesh=_mesh(),
        scratch_types=[pltpu.VMEM((CPT, CHUNK), jnp.int32),
                       pltpu.VMEM((8, CHUNK, w), jnp.float32),
                       pltpu.VMEM_SHARED((N_ACC, w), jnp.float32),
                       pltpu.SemaphoreType.DMA,
                       pltpu.SemaphoreType.DMA],
        compiler_params=pltpu.CompilerParams(use_tc_tiling_on_sc=False),
    )
    def scat(h_hbm, idx3, z_hbm, out0, out1, idxv, hbuf, acc, lsem, asem):
        cid = lax.axis_index("c")
        sid = lax.axis_index("s")
        wid = sid * NC + cid
        base = wid * EPT
        sl = pl.ds(sid * ROWS_PT, ROWS_PT)
        pltpu.sync_copy(z_hbm, acc.at[sl])
        plsc.subcore_barrier()
        pltpu.sync_copy(idx3.at[wid], idxv)
        nb = 8

        def group(k, c):
            descs = []
            for b in range(nb):
                j = k * nb + b
                descs.append(pltpu.async_copy(
                    h_hbm.at[pl.ds(base + j * CHUNK, CHUNK)], hbuf.at[b],
                    lsem))
            for d in descs:
                d.wait()
            descs = []
            for b in range(nb):
                j = k * nb + b
                descs.append(pltpu.async_copy(
                    hbuf.at[b], acc.at[idxv.at[j]], asem, add=True))
            for d in descs:
                d.wait()
            return c

        lax.fori_loop(0, CPT // nb, group, 0)
        plsc.subcore_barrier()

        @pl.when(cid == 0)
        def _():
            pltpu.sync_copy(acc.at[sl], out0.at[sl])

        @pl.when(cid == 1)
        def _():
            pltpu.sync_copy(acc.at[sl], out1.at[sl])
    return scat


@functools.lru_cache(maxsize=None)
def _sc_degree():
    @functools.partial(
        pl.kernel,
        out_type=(jax.ShapeDtypeStruct((N_ACC, 16), jnp.float32),
                  jax.ShapeDtypeStruct((N_ACC, 16), jnp.float32)),
        mesh=_mesh(),
        scratch_types=[pltpu.VMEM((CPT, CHUNK), jnp.int32),
                       pltpu.VMEM((CHUNK, 16), jnp.float32),
                       pltpu.VMEM_SHARED((N_ACC, 16), jnp.float32),
                       pltpu.SemaphoreType.DMA],
        compiler_params=pltpu.CompilerParams(use_tc_tiling_on_sc=False),
    )
    def degk(idx3, z_hbm, ones_hbm, out0, out1, idxv, obuf, acc, asem):
        cid = lax.axis_index("c")
        sid = lax.axis_index("s")
        wid = sid * NC + cid
        sl = pl.ds(sid * ROWS_PT, ROWS_PT)
        pltpu.sync_copy(z_hbm, acc.at[sl])
        pltpu.sync_copy(ones_hbm, obuf)
        plsc.subcore_barrier()
        pltpu.sync_copy(idx3.at[wid], idxv)
        nb = 8

        def group(k, c):
            descs = [pltpu.async_copy(obuf, acc.at[idxv.at[k * nb + b]],
                                      asem, add=True) for b in range(nb)]
            for d in descs:
                d.wait()
            return c

        lax.fori_loop(0, CPT // nb, group, 0)
        plsc.subcore_barrier()

        @pl.when(cid == 0)
        def _():
            pltpu.sync_copy(acc.at[sl], out0.at[sl])

        @pl.when(cid == 1)
        def _():
            pltpu.sync_copy(acc.at[sl], out1.at[sl])
    return degk


def _stats(x, bs, want_cast=False):
    m, c = x.shape
    nb = m // bs

    def body(x_ref, o_ref, *rest):
        i = pl.program_id(0)

        @pl.when(i == 0)
        def _():
            o_ref[...] = jnp.zeros_like(o_ref)

        xb = x_ref[...]
        o_ref[0:1, :] += jnp.sum(xb, axis=0, keepdims=True)
        o_ref[1:2, :] += jnp.sum(xb * xb, axis=0, keepdims=True)
        if want_cast:
            rest[0][...] = xb.astype(jnp.bfloat16)

    out_specs = [pl.BlockSpec((8, c), lambda i: (0, 0))]
    out_shape = [jax.ShapeDtypeStruct((8, c), jnp.float32)]
    if want_cast:
        out_specs.append(pl.BlockSpec((bs, c), lambda i: (i, 0)))
        out_shape.append(jax.ShapeDtypeStruct((m, c), jnp.bfloat16))
    out = pl.pallas_call(
        body,
        grid=(nb,),
        in_specs=[pl.BlockSpec((bs, c), lambda i: (i, 0))],
        out_specs=out_specs,
        out_shape=out_shape,
    )(x)
    if want_cast:
        return out[0][0], out[0][1], out[1]
    return out[0][0], out[0][1]


def _edge_call(nf, ef, xr, xc, e, w1a, w1b, w1c, b1, wh, bh,
               wna, wn2, bn2, wp, bp):
    def body(*refs):
        (xr_ref, xc_ref, e_ref) = refs[:3]
        k = 3
        if w1c is not None:
            w1c_r = refs[k]
            k += 1
        (w1a_r, w1b_r, b1_r, wh_r, bh_r,
         wna_r, wn2_r, bn2_r) = refs[k:k + 8]
        k += 8
        if wp is not None:
            wp_r, bp_r = refs[k:k + 2]
            k += 2
        oh_ref = refs[k]
        xrv = xr_ref[...].astype(jnp.float32)
        xcv = xc_ref[...].astype(jnp.float32)
        ev = e_ref[...]
        z = _dot(xrv, w1a_r[...]) + _dot(xcv, w1b_r[...]) + b1_r[...]
        if w1c is not None:
            z = z + _dot(ev, w1c_r[...])
        else:
            z = z + ev
        t1 = _elu(z)
        if wp is not None:
            refs[k + 1][...] = _dot(t1, wp_r[...]) + bp_r[...]
        t2 = _elu(_dot(xrv, wna_r[...]) + _dot(t1, wh_r[...]) + bh_r[...])
        oh_ref[...] = _dot(t2, wn2_r[...]) + bn2_r[...]

    ws = []
    if w1c is not None:
        ws.append(w1c)
    ws += [w1a, w1b, b1, wh, bh, wna, wn2, bn2]
    if wp is not None:
        ws += [wp, bp]
    out_specs = [pl.BlockSpec((BE, 64), lambda i: (i, 0))]
    out_shape = [jax.ShapeDtypeStruct((E_PAD, 64), jnp.float32)]
    if wp is not None:
        out_specs.append(pl.BlockSpec((BE, 32), lambda i: (i, 0)))
        out_shape.append(jax.ShapeDtypeStruct((E_PAD, 32), jnp.float32))
    out = pl.pallas_call(
        body,
        grid=(E_PAD // BE,),
        in_specs=[pl.BlockSpec((BE, nf), lambda i: (i, 0)),
                  pl.BlockSpec((BE, nf), lambda i: (i, 0)),
                  pl.BlockSpec((BE, ef), lambda i: (i, 0))]
                 + [_full_spec(a.shape) for a in ws],
        out_specs=out_specs,
        out_shape=out_shape,
        compiler_params=pltpu.CompilerParams(
            dimension_semantics=("arbitrary",)),
    )(xr, xc, e, *ws)
    if wp is not None:
        return out[1], out[0]
    return None, out[0]


def _node_call(nf, gout, has_u, x, hs0, hs1, dg0, dg1, batch2, u,
               w21a, w21b, b21, w22, b22, wg1u, wg1g, bg1, wg2, bg2):
    def body(*refs):
        (x_ref, hs0_ref, hs1_ref, dg0_ref, dg1_ref, batch_ref) = refs[:6]
        k = 6
        if has_u:
            u_ref = refs[k]
            k += 1
        (w21a_r, w21b_r, b21_r, w22_r, b22_r) = refs[k:k + 5]
        k += 5
        if has_u:
            wg1u_r = refs[k]
            k += 1
        (wg1g_r, bg1_r, wg2_r, bg2_r) = refs[k:k + 4]
        k += 4
        xn_ref, un_ref = refs[k:k + 2]
        acc = refs[k + 2]
        i = pl.program_id(0)

        @pl.when(i < NBLK)
        def _():
            hsv = hs0_ref[...] + hs1_ref[...]
            degv = dg0_ref[:, 0:1] + dg1_ref[:, 0:1]
            hm = hsv / jnp.maximum(degv, 1.0)
            xv = x_ref[...].astype(jnp.float32)
            t = _elu(_dot(xv, w21a_r[...]) + _dot(hm, w21b_r[...])
                     + b21_r[...])
            xn = _dot(t, w22_r[...]) + b22_r[...]
            xn_ref[...] = xn.astype(jnp.bfloat16)
            bb = batch_ref[...].reshape(1, BNODE)
            gids = lax.broadcasted_iota(jnp.int32, (G, BNODE), 0)
            oh = (gids == bb).astype(jnp.float32)
            ones = jnp.ones((BNODE, 16), jnp.float32)
            contrib = _dot(oh, jnp.concatenate([xn, ones], axis=1))

            @pl.when(i == 0)
            def _():
                acc[...] = jnp.zeros_like(acc)

            acc[...] += contrib

        @pl.when(i == NBLK)
        def _():
            cnt = acc[:, 32:33]
            gm = acc[:, 0:32] / jnp.maximum(cnt, 1.0)
            z = _dot(gm, wg1g_r[...]) + bg1_r[...]
            if has_u:
                z = z + _dot(u_ref[...], wg1u_r[...])
            tg = _elu(z)
            un_ref[...] = _dot(tg, wg2_r[...]) + bg2_r[...]

    jcap = lambda i: (jnp.minimum(i, NBLK - 1), 0)
    jcap3 = lambda i: (jnp.minimum(i, NBLK - 1), 0)
    in_arrays = [x, hs0, hs1, dg0, dg1, batch2]
    in_specs = [pl.BlockSpec((BNODE, nf), jcap),
                pl.BlockSpec((BNODE, 64), jcap3),
                pl.BlockSpec((BNODE, 64), jcap3),
                pl.BlockSpec((BNODE, 16), jcap3),
                pl.BlockSpec((BNODE, 16), jcap3),
                pl.BlockSpec((BNODE, 1), jcap)]
    if has_u:
        in_arrays.append(u)
        in_specs.append(_full_spec(u.shape))
    ws = [w21a, w21b, b21, w22, b22]
    if has_u:
        ws.append(wg1u)
    ws += [wg1g, bg1, wg2, bg2]
    in_arrays += ws
    in_specs += [_full_spec(a.shape) for a in ws]
    return pl.pallas_call(
        body,
        grid=(NBLK + 1,),
        in_specs=in_specs,
        out_specs=[pl.BlockSpec((BNODE, 32), jcap),
                   pl.BlockSpec((G, gout), lambda i: (0, 0))],
        out_shape=[jax.ShapeDtypeStruct((N, 32), jnp.bfloat16),
                   jax.ShapeDtypeStruct((G, gout), jnp.float32)],
        scratch_shapes=[pltpu.VMEM((G, 48), jnp.float32)],
        compiler_params=pltpu.CompilerParams(
            dimension_semantics=("arbitrary",)),
    )(*in_arrays)


def _final_call(u, w1, b1, w2, b2):
    def body(u_ref, w1_r, b1_r, w2_r, b2_r, o_ref):
        t = _elu(_dot(u_ref[...], w1_r[...]) + b1_r[...])
        o_ref[...] = _dot(t, w2_r[...]) + b2_r[...]

    return pl.pallas_call(
        body,
        out_shape=jax.ShapeDtypeStruct((G, 256), jnp.float32),
    )(u, w1, b1, w2, b2)


def _b2(b):
    return b.reshape(1, -1)


def kernel(x, edge_attr, params, edge_index, batch):
    p = params
    row = edge_index[0]
    col = edge_index[1]
    zpad = jnp.zeros((PAD,), jnp.int32)
    ridx3 = jnp.concatenate([row, zpad]).reshape(TILES, CPT, CHUNK)
    cidx3 = jnp.concatenate([col, zpad]).reshape(TILES, CPT, CHUNK)
    sidx3 = jnp.concatenate(
        [col, jnp.full((PAD,), N, jnp.int32)]).reshape(TILES, CPT, CHUNK)
    e0 = jnp.concatenate(
        [edge_attr, jnp.zeros((PAD, DE), jnp.float32)], axis=0)
    zeros64 = jnp.zeros((ROWS_PT, 64), jnp.float32)
    zeros16 = jnp.zeros((ROWS_PT, 16), jnp.float32)
    ones16 = jnp.ones((CHUNK, 16), jnp.float32)
    batch2 = batch.reshape(N, 1)

    dg0, dg1 = _sc_degree()(sidx3, zeros16, ones16)

    sx, qx, x16 = _stats(x, BNODE, want_cast=True)
    se, qe = _stats(edge_attr, 8000)
    mx = sx / N
    vx = qx / N - mx * mx
    me = se / E
    ve = qe / E - me * me
    s_x = p["bn_node"]["g"] / jnp.sqrt(vx + 1e-5)
    t_x = p["bn_node"]["b"] - mx * s_x
    s_e = p["bn_edge"]["g"] / jnp.sqrt(ve + 1e-5)
    t_e = p["bn_edge"]["b"] - me * s_e

    m1 = p["m1"]
    e1w, e1b = m1["e1"]["w"], m1["e1"]["b"]
    w1a = e1w[:DN] * s_x[:, None]
    w1b = e1w[DN:2 * DN] * s_x[:, None]
    w1c = e1w[2 * DN:] * s_e[:, None]
    b1 = (e1b + t_x @ e1w[:DN] + t_x @ e1w[DN:2 * DN] + t_e @ e1w[2 * DN:])
    n11w, n11b = m1["n11"]["w"], m1["n11"]["b"]
    wna = n11w[:DN] * s_x[:, None]
    wnb = n11w[DN:]
    bn1 = n11b + t_x @ n11w[:DN]
    n21w, n21b = m1["n21"]["w"], m1["n21"]["b"]
    w21a = n21w[:DN] * s_x[:, None]
    w21b = n21w[DN:]
    b21 = n21b + t_x @ n21w[:DN]

    w2m1, b2m1 = m1["e2"]["w"], m1["e2"]["b"]
    w1cn = p["m2"]["e1"]["w"][64:]
    xr, xc = _sc_gather(DN)(x16, ridx3, cidx3)
    ecur, h = _edge_call(
        DN, DE, xr, xc, e0,
        w1a, w1b, w1c, _b2(b1), w2m1 @ wnb, _b2(bn1 + b2m1 @ wnb),
        wna, m1["n12"]["w"], _b2(m1["n12"]["b"]),
        w2m1 @ w1cn, _b2(b2m1 @ w1cn))
    hs0, hs1 = _sc_scatter(64)(h, sidx3, zeros64)
    xcur, u = _node_call(
        DN, 32, False, x, hs0, hs1, dg0, dg1, batch2, None,
        w21a, w21b, _b2(b21), m1["n22"]["w"], _b2(m1["n22"]["b"]),
        None, m1["g1"]["w"], _b2(m1["g1"]["b"]),
        m1["g2"]["w"], _b2(m1["g2"]["b"]))

    names = ("m2", "m3", "m4", "m5", "m6", "m7")
    for li, name in enumerate(names):
        mp = p[name]
        gout = mp["g2"]["w"].shape[1]
        e1w = mp["e1"]["w"]
        n11w = mp["n11"]["w"]
        n21w = mp["n21"]["w"]
        g1w = mp["g1"]["w"]
        w2, b2 = mp["e2"]["w"], mp["e2"]["b"]
        wnb = n11w[32:]
        if li + 1 < len(names):
            w1cn = p[names[li + 1]]["e1"]["w"][64:]
            wp, bp = w2 @ w1cn, _b2(b2 @ w1cn)
        else:
            wp = bp = None
        xr, xc = _sc_gather(32)(xcur, ridx3, cidx3)
        ecur, h = _edge_call(
            32, 32, xr, xc, ecur,
            e1w[:32], e1w[32:64], None, _b2(mp["e1"]["b"]),
            w2 @ wnb, _b2(mp["n11"]["b"] + b2 @ wnb),
            n11w[:32], mp["n12"]["w"], _b2(mp["n12"]["b"]), wp, bp)
        hs0, hs1 = _sc_scatter(64)(h, sidx3, zeros64)
        xcur, u = _node_call(
            32, gout, True, xcur, hs0, hs1, dg0, dg1, batch2, u,
            n21w[:32], n21w[32:], _b2(mp["n21"]["b"]),
            mp["n22"]["w"], _b2(mp["n22"]["b"]),
            g1w[:32], g1w[32:], _b2(mp["g1"]["b"]),
            mp["g2"]["w"], _b2(mp["g2"]["b"]))

    return _final_call(u, p["lin1"]["w"], _b2(p["lin1"]["b"]),
                       p["lin2"]["w"], _b2(p["lin2"]["b"]))


# speedup vs baseline: 1.1749x; 1.0257x over previous
import functools

import jax
import jax.numpy as jnp
from jax import lax
from jax.experimental import pallas as pl
from jax.experimental.pallas import tpu as pltpu
from jax.experimental.pallas import tpu_sc as plsc

N = 10000
E = 320000
DN = 128
DE = 16
G = 256

NC, NS = 2, 16
TILES = NC * NS
CHUNK = 128
CPT = 80
EPT = CHUNK * CPT
E_PAD = TILES * EPT
PAD = E_PAD - E
ROWS_PT = 632
N_ACC = ROWS_PT * NS
BE = 2048
BNODE = 1000
NBLK = N // BNODE

@functools.lru_cache(maxsize=None)
def _mesh():
    return plsc.VectorSubcoreMesh(core_axis_name="c", subcore_axis_name="s",
                                  num_cores=NC, num_subcores=NS)


def _elu(v):
    return jnp.where(v > 0, v, jnp.exp(jnp.minimum(v, 0.0)) - 1.0)


def _dot(a, b):
    return jnp.dot(a, b, preferred_element_type=jnp.float32)


def _full_spec(shape):
    n = len(shape)
    return pl.BlockSpec(shape, lambda i, _n=n: (0,) * _n)


@functools.lru_cache(maxsize=None)
def _sc_gather(nf, cpt=CPT):
    ept = cpt * CHUNK
    ne = TILES * ept
    nb = 2 if nf > 64 else 4
    ng = cpt // nb
    @functools.partial(
        pl.kernel,
        out_type=(jax.ShapeDtypeStruct((ne, nf), jnp.bfloat16),
                  jax.ShapeDtypeStruct((ne, nf), jnp.bfloat16)),
        mesh=_mesh(),
        scratch_types=[pltpu.VMEM((cpt, CHUNK), jnp.int32),
                       pltpu.VMEM((cpt, CHUNK), jnp.int32),
                       pltpu.VMEM((nb, CHUNK, nf), jnp.bfloat16),
                       pltpu.VMEM((nb, CHUNK, nf), jnp.bfloat16),
                       pltpu.SemaphoreType.DMA,
                       pltpu.SemaphoreType.DMA],
        compiler_params=pltpu.CompilerParams(use_tc_tiling_on_sc=False),
    )
    def gath(x_hbm, ridx, cidx, xr_out, xc_out, idxr, idxc, bufr, bufc,
             gsem, ssem):
        cid = lax.axis_index("c")
        sid = lax.axis_index("s")
        wid = sid * NC + cid
        base = wid * ept
        pltpu.sync_copy(ridx.at[wid], idxr)
        pltpu.sync_copy(cidx.at[wid], idxc)

        def group(k, c):
            descs = []
            for b in range(nb):
                j = k * nb + b
                descs.append(
                    pltpu.async_copy(x_hbm.at[idxr.at[j]], bufr.at[b], gsem))
                descs.append(
                    pltpu.async_copy(x_hbm.at[idxc.at[j]], bufc.at[b], gsem))
            for d in descs:
                d.wait()
            descs = []
            for b in range(nb):
                j = k * nb + b
                dst = pl.ds(base + j * CHUNK, CHUNK)
                descs.append(
                    pltpu.async_copy(bufr.at[b], xr_out.at[dst], ssem))
                descs.append(
                    pltpu.async_copy(bufc.at[b], xc_out.at[dst], ssem))
            for d in descs:
                d.wait()
            return c

        lax.fori_loop(0, ng, group, 0)
    return gath


@functools.lru_cache(maxsize=None)
def _sc_scatter(w, cpt=CPT):
    ept = cpt * CHUNK
    @functools.partial(
        pl.kernel,
        out_type=(jax.ShapeDtypeStruct((N_ACC, w), jnp.float32),
                  jax.ShapeDtypeStruct((N_ACC, w), jnp.float32)),
        mesh=_mesh(),
        scratch_types=[pltpu.VMEM((cpt, CHUNK), jnp.int32),
                       pltpu.VMEM((8, CHUNK, w), jnp.float32),
                       pltpu.VMEM_SHARED((N_ACC, w), jnp.float32),
                       pltpu.SemaphoreType.DMA,
                       pltpu.SemaphoreType.DMA],
        compiler_params=pltpu.CompilerParams(use_tc_tiling_on_sc=False),
    )
    def scat(h_hbm, idx3, z_hbm, out0, out1, idxv, hbuf, acc, lsem, asem):
        cid = lax.axis_index("c")
        sid = lax.axis_index("s")
        wid = sid * NC + cid
        base = wid * ept
        sl = pl.ds(sid * ROWS_PT, ROWS_PT)
        pltpu.sync_copy(z_hbm, acc.at[sl])
        plsc.subcore_barrier()
        pltpu.sync_copy(idx3.at[wid], idxv)
        nb = 8

        def group(k, c):
            descs = []
            for b in range(nb):
                j = k * nb + b
                descs.append(pltpu.async_copy(
                    h_hbm.at[pl.ds(base + j * CHUNK, CHUNK)], hbuf.at[b],
                    lsem))
            for d in descs:
                d.wait()
            descs = []
            for b in range(nb):
                j = k * nb + b
                descs.append(pltpu.async_copy(
                    hbuf.at[b], acc.at[idxv.at[j]], asem, add=True))
            for d in descs:
                d.wait()
            return c

        lax.fori_loop(0, cpt // nb, group, 0)
        plsc.subcore_barrier()

        @pl.when(cid == 0)
        def _():
            pltpu.sync_copy(acc.at[sl], out0.at[sl])

        @pl.when(cid == 1)
        def _():
            pltpu.sync_copy(acc.at[sl], out1.at[sl])
    return scat


@functools.lru_cache(maxsize=None)
def _sc_degree():
    @functools.partial(
        pl.kernel,
        out_type=(jax.ShapeDtypeStruct((N_ACC, 16), jnp.float32),
                  jax.ShapeDtypeStruct((N_ACC, 16), jnp.float32)),
        mesh=_mesh(),
        scratch_types=[pltpu.VMEM((CPT, CHUNK), jnp.int32),
                       pltpu.VMEM((CHUNK, 16), jnp.float32),
                       pltpu.VMEM_SHARED((N_ACC, 16), jnp.float32),
                       pltpu.SemaphoreType.DMA],
        compiler_params=pltpu.CompilerParams(use_tc_tiling_on_sc=False),
    )
    def degk(idx3, z_hbm, ones_hbm, out0, out1, idxv, obuf, acc, asem):
        cid = lax.axis_index("c")
        sid = lax.axis_index("s")
        wid = sid * NC + cid
        sl = pl.ds(sid * ROWS_PT, ROWS_PT)
        pltpu.sync_copy(z_hbm, acc.at[sl])
        pltpu.sync_copy(ones_hbm, obuf)
        plsc.subcore_barrier()
        pltpu.sync_copy(idx3.at[wid], idxv)
        nb = 8

        def group(k, c):
            descs = [pltpu.async_copy(obuf, acc.at[idxv.at[k * nb + b]],
                                      asem, add=True) for b in range(nb)]
            for d in descs:
                d.wait()
            return c

        lax.fori_loop(0, CPT // nb, group, 0)
        plsc.subcore_barrier()

        @pl.when(cid == 0)
        def _():
            pltpu.sync_copy(acc.at[sl], out0.at[sl])

        @pl.when(cid == 1)
        def _():
            pltpu.sync_copy(acc.at[sl], out1.at[sl])
    return degk


def _stats(x, bs, want_cast=False):
    m, c = x.shape
    nb = m // bs

    def body(x_ref, o_ref, *rest):
        i = pl.program_id(0)

        @pl.when(i == 0)
        def _():
            o_ref[...] = jnp.zeros_like(o_ref)

        xb = x_ref[...]
        o_ref[0:1, :] += jnp.sum(xb, axis=0, keepdims=True)
        o_ref[1:2, :] += jnp.sum(xb * xb, axis=0, keepdims=True)
        if want_cast:
            rest[0][...] = xb.astype(jnp.bfloat16)

    out_specs = [pl.BlockSpec((8, c), lambda i: (0, 0))]
    out_shape = [jax.ShapeDtypeStruct((8, c), jnp.float32)]
    if want_cast:
        out_specs.append(pl.BlockSpec((bs, c), lambda i: (i, 0)))
        out_shape.append(jax.ShapeDtypeStruct((m, c), jnp.bfloat16))
    out = pl.pallas_call(
        body,
        grid=(nb,),
        in_specs=[pl.BlockSpec((bs, c), lambda i: (i, 0))],
        out_specs=out_specs,
        out_shape=out_shape,
    )(x)
    if want_cast:
        return out[0][0], out[0][1], out[1]
    return out[0][0], out[0][1]


def _edge_call(nf, ef, ne, xr, xc, e, w1a, w1b, w1c, b1, wh, bh,
               wna, wn2, bn2, wp, bp):
    def body(*refs):
        (xr_ref, xc_ref, e_ref) = refs[:3]
        k = 3
        if w1c is not None:
            w1c_r = refs[k]
            k += 1
        (w1a_r, w1b_r, b1_r, wh_r, bh_r,
         wna_r, wn2_r, bn2_r) = refs[k:k + 8]
        k += 8
        if wp is not None:
            wp_r, bp_r = refs[k:k + 2]
            k += 2
        oh_ref = refs[k]
        xrv = xr_ref[...].astype(jnp.float32)
        xcv = xc_ref[...].astype(jnp.float32)
        ev = e_ref[...]
        z = _dot(xrv, w1a_r[...]) + _dot(xcv, w1b_r[...]) + b1_r[...]
        if w1c is not None:
            z = z + _dot(ev, w1c_r[...])
        else:
            z = z + ev
        t1 = _elu(z)
        if wp is not None:
            refs[k + 1][...] = _dot(t1, wp_r[...]) + bp_r[...]
        t2 = _elu(_dot(xrv, wna_r[...]) + _dot(t1, wh_r[...]) + bh_r[...])
        oh_ref[...] = _dot(t2, wn2_r[...]) + bn2_r[...]

    ws = []
    if w1c is not None:
        ws.append(w1c)
    ws += [w1a, w1b, b1, wh, bh, wna, wn2, bn2]
    if wp is not None:
        ws += [wp, bp]
    out_specs = [pl.BlockSpec((BE, 64), lambda i: (i, 0))]
    out_shape = [jax.ShapeDtypeStruct((ne, 64), jnp.float32)]
    if wp is not None:
        out_specs.append(pl.BlockSpec((BE, 32), lambda i: (i, 0)))
        out_shape.append(jax.ShapeDtypeStruct((ne, 32), jnp.float32))
    out = pl.pallas_call(
        body,
        grid=(ne // BE,),
        in_specs=[pl.BlockSpec((BE, nf), lambda i: (i, 0)),
                  pl.BlockSpec((BE, nf), lambda i: (i, 0)),
                  pl.BlockSpec((BE, ef), lambda i: (i, 0))]
                 + [_full_spec(a.shape) for a in ws],
        out_specs=out_specs,
        out_shape=out_shape,
        compiler_params=pltpu.CompilerParams(
            dimension_semantics=("arbitrary",)),
    )(xr, xc, e, *ws)
    if wp is not None:
        return out[1], out[0]
    return None, out[0]


def _node_call(nf, gout, has_u, x, hss, dg0, dg1, batch2, u,
               w21a, w21b, b21, w22, b22, wg1u, wg1g, bg1, wg2, bg2):
    nhs = len(hss)

    def body(*refs):
        x_ref = refs[0]
        hs_refs = refs[1:1 + nhs]
        (dg0_ref, dg1_ref, batch_ref) = refs[1 + nhs:4 + nhs]
        k = 4 + nhs
        if has_u:
            u_ref = refs[k]
            k += 1
        (w21a_r, w21b_r, b21_r, w22_r, b22_r) = refs[k:k + 5]
        k += 5
        if has_u:
            wg1u_r = refs[k]
            k += 1
        (wg1g_r, bg1_r, wg2_r, bg2_r) = refs[k:k + 4]
        k += 4
        xn_ref, un_ref = refs[k:k + 2]
        acc = refs[k + 2]
        i = pl.program_id(0)

        @pl.when(i < NBLK)
        def _():
            hsv = hs_refs[0][...]
            for r in hs_refs[1:]:
                hsv = hsv + r[...]
            degv = dg0_ref[:, 0:1] + dg1_ref[:, 0:1]
            hm = hsv / jnp.maximum(degv, 1.0)
            xv = x_ref[...].astype(jnp.float32)
            t = _elu(_dot(xv, w21a_r[...]) + _dot(hm, w21b_r[...])
                     + b21_r[...])
            xn = _dot(t, w22_r[...]) + b22_r[...]
            xn_ref[...] = xn.astype(jnp.bfloat16)
            bb = batch_ref[...].reshape(1, BNODE)
            gids = lax.broadcasted_iota(jnp.int32, (G, BNODE), 0)
            oh = (gids == bb).astype(jnp.float32)
            ones = jnp.ones((BNODE, 16), jnp.float32)
            contrib = _dot(oh, jnp.concatenate([xn, ones], axis=1))

            @pl.when(i == 0)
            def _():
                acc[...] = jnp.zeros_like(acc)

            acc[...] += contrib

        @pl.when(i == NBLK)
        def _():
            cnt = acc[:, 32:33]
            gm = acc[:, 0:32] / jnp.maximum(cnt, 1.0)
            z = _dot(gm, wg1g_r[...]) + bg1_r[...]
            if has_u:
                z = z + _dot(u_ref[...], wg1u_r[...])
            tg = _elu(z)
            un_ref[...] = _dot(tg, wg2_r[...]) + bg2_r[...]

    jcap = lambda i: (jnp.minimum(i, NBLK - 1), 0)
    jcap3 = lambda i: (jnp.minimum(i, NBLK - 1), 0)
    in_arrays = [x] + list(hss) + [dg0, dg1, batch2]
    in_specs = ([pl.BlockSpec((BNODE, nf), jcap)]
                + [pl.BlockSpec((BNODE, 64), jcap3)] * nhs
                + [pl.BlockSpec((BNODE, 16), jcap3),
                   pl.BlockSpec((BNODE, 16), jcap3),
                   pl.BlockSpec((BNODE, 1), jcap)])
    if has_u:
        in_arrays.append(u)
        in_specs.append(_full_spec(u.shape))
    ws = [w21a, w21b, b21, w22, b22]
    if has_u:
        ws.append(wg1u)
    ws += [wg1g, bg1, wg2, bg2]
    in_arrays += ws
    in_specs += [_full_spec(a.shape) for a in ws]
    return pl.pallas_call(
        body,
        grid=(NBLK + 1,),
        in_specs=in_specs,
        out_specs=[pl.BlockSpec((BNODE, 32), jcap),
                   pl.BlockSpec((G, gout), lambda i: (0, 0))],
        out_shape=[jax.ShapeDtypeStruct((N, 32), jnp.bfloat16),
                   jax.ShapeDtypeStruct((G, gout), jnp.float32)],
        scratch_shapes=[pltpu.VMEM((G, 48), jnp.float32)],
        compiler_params=pltpu.CompilerParams(
            dimension_semantics=("arbitrary",)),
    )(*in_arrays)


def _final_call(u, w1, b1, w2, b2):
    def body(u_ref, w1_r, b1_r, w2_r, b2_r, o_ref):
        t = _elu(_dot(u_ref[...], w1_r[...]) + b1_r[...])
        o_ref[...] = _dot(t, w2_r[...]) + b2_r[...]

    return pl.pallas_call(
        body,
        out_shape=jax.ShapeDtypeStruct((G, 256), jnp.float32),
    )(u, w1, b1, w2, b2)


def _b2(b):
    return b.reshape(1, -1)


def kernel(x, edge_attr, params, edge_index, batch):
    p = params
    row = edge_index[0]
    col = edge_index[1]
    zpad = jnp.zeros((PAD,), jnp.int32)
    cpt_h = CPT // 2
    eh = E_PAD // 2
    rfull = jnp.concatenate([row, zpad])
    cfull = jnp.concatenate([col, zpad])
    sfull = jnp.concatenate([col, jnp.full((PAD,), N, jnp.int32)])
    sidx3 = sfull.reshape(TILES, CPT, CHUNK)
    ridxh = [rfull[i * eh:(i + 1) * eh].reshape(TILES, cpt_h, CHUNK)
             for i in range(2)]
    cidxh = [cfull[i * eh:(i + 1) * eh].reshape(TILES, cpt_h, CHUNK)
             for i in range(2)]
    sidxh = [sfull[i * eh:(i + 1) * eh].reshape(TILES, cpt_h, CHUNK)
             for i in range(2)]
    e0 = jnp.concatenate(
        [edge_attr, jnp.zeros((PAD, DE), jnp.float32)], axis=0)
    e0h = [e0[i * eh:(i + 1) * eh] for i in range(2)]
    zeros64 = jnp.zeros((ROWS_PT, 64), jnp.float32)
    zeros16 = jnp.zeros((ROWS_PT, 16), jnp.float32)
    ones16 = jnp.ones((CHUNK, 16), jnp.float32)
    batch2 = batch.reshape(N, 1)

    dg0, dg1 = _sc_degree()(sidx3, zeros16, ones16)

    sx, qx, x16 = _stats(x, BNODE, want_cast=True)
    se, qe = _stats(edge_attr, 8000)
    mx = sx / N
    vx = qx / N - mx * mx
    me = se / E
    ve = qe / E - me * me
    s_x = p["bn_node"]["g"] / jnp.sqrt(vx + 1e-5)
    t_x = p["bn_node"]["b"] - mx * s_x
    s_e = p["bn_edge"]["g"] / jnp.sqrt(ve + 1e-5)
    t_e = p["bn_edge"]["b"] - me * s_e

    m1 = p["m1"]
    e1w, e1b = m1["e1"]["w"], m1["e1"]["b"]
    w1a = e1w[:DN] * s_x[:, None]
    w1b = e1w[DN:2 * DN] * s_x[:, None]
    w1c = e1w[2 * DN:] * s_e[:, None]
    b1 = (e1b + t_x @ e1w[:DN] + t_x @ e1w[DN:2 * DN] + t_e @ e1w[2 * DN:])
    n11w, n11b = m1["n11"]["w"], m1["n11"]["b"]
    wna = n11w[:DN] * s_x[:, None]
    wnb = n11w[DN:]
    bn1 = n11b + t_x @ n11w[:DN]
    n21w, n21b = m1["n21"]["w"], m1["n21"]["b"]
    w21a = n21w[:DN] * s_x[:, None]
    w21b = n21w[DN:]
    b21 = n21b + t_x @ n21w[:DN]

    w2m1, b2m1 = m1["e2"]["w"], m1["e2"]["b"]
    w1cn = p["m2"]["e1"]["w"][64:]
    ew1 = (w1a, w1b, w1c, _b2(b1), w2m1 @ wnb, _b2(bn1 + b2m1 @ wnb),
           wna, m1["n12"]["w"], _b2(m1["n12"]["b"]),
           w2m1 @ w1cn, _b2(b2m1 @ w1cn))
    ecur = [None, None]
    hs = []
    xrh = [None, None]
    xch = [None, None]
    for i in range(2):
        xrh[i], xch[i] = _sc_gather(DN, cpt_h)(x16, ridxh[i], cidxh[i])
    for i in range(2):
        ecur[i], h = _edge_call(DN, DE, eh, xrh[i], xch[i], e0h[i], *ew1)
        hs += list(_sc_scatter(64, cpt_h)(h, sidxh[i], zeros64))
    xcur, u = _node_call(
        DN, 32, False, x, tuple(hs), dg0, dg1, batch2, None,
        w21a, w21b, _b2(b21), m1["n22"]["w"], _b2(m1["n22"]["b"]),
        None, m1["g1"]["w"], _b2(m1["g1"]["b"]),
        m1["g2"]["w"], _b2(m1["g2"]["b"]))

    names = ("m2", "m3", "m4", "m5", "m6", "m7")
    for li, name in enumerate(names):
        mp = p[name]
        gout = mp["g2"]["w"].shape[1]
        e1w = mp["e1"]["w"]
        n11w = mp["n11"]["w"]
        n21w = mp["n21"]["w"]
        g1w = mp["g1"]["w"]
        w2, b2 = mp["e2"]["w"], mp["e2"]["b"]
        wnb = n11w[32:]
        if li + 1 < len(names):
            w1cn = p[names[li + 1]]["e1"]["w"][64:]
            wp, bp = w2 @ w1cn, _b2(b2 @ w1cn)
        else:
            wp = bp = None
        ew = (e1w[:32], e1w[32:64], None, _b2(mp["e1"]["b"]),
              w2 @ wnb, _b2(mp["n11"]["b"] + b2 @ wnb),
              n11w[:32], mp["n12"]["w"], _b2(mp["n12"]["b"]), wp, bp)
        enew = [None, None]
        hs = []
        for i in range(2):
            xrh[i], xch[i] = _sc_gather(32, cpt_h)(xcur, ridxh[i], cidxh[i])
        for i in range(2):
            enew[i], h = _edge_call(32, 32, eh, xrh[i], xch[i], ecur[i], *ew)
            hs += list(_sc_scatter(64, cpt_h)(h, sidxh[i], zeros64))
        ecur = enew
        xcur, u = _node_call(
            32, gout, True, xcur, tuple(hs), dg0, dg1, batch2, u,
            n21w[:32], n21w[32:], _b2(mp["n21"]["b"]),
            mp["n22"]["w"], _b2(mp["n22"]["b"]),
            g1w[:32], g1w[32:], _b2(mp["g1"]["b"]),
            mp["g2"]["w"], _b2(mp["g2"]["b"]))

    return _final_call(u, p["lin1"]["w"], _b2(p["lin1"]["b"]),
                       p["lin2"]["w"], _b2(p["lin2"]["b"]))
